# baseline (speedup 1.0000x reference)
import functools

import jax
import jax.numpy as jnp
from jax import lax
from jax.experimental import pallas as pl
from jax.experimental.pallas import tpu as pltpu
from jax.experimental.pallas import tpu_sc as plsc

N = 10000
E = 160000
D = 36
DE = 6
EH = 36
STEPS = 3

DP = 48
NP = 10240
EP = 163840
NH = 2
EPH = EP // NH
NC = 2
NS = 16
NW = NC * NS
RL = 128
EPW = EPH // NW
ROWS = EPW // RL
GR = 10
NCH = ROWS // GR
NSTRIPE = NP // NS

TE = 1280
TN = 640
DPB = 64


@functools.lru_cache(maxsize=None)
def _mesh():
    return plsc.VectorSubcoreMesh(core_axis_name="c", subcore_axis_name="s")


def _gather_body(table_hbm, idx_hbm, out_hbm, idx_v, rows_v, sem):
    cid = lax.axis_index("c")
    sid = lax.axis_index("s")
    wid = sid * NC + cid
    base = wid * EPW
    pltpu.sync_copy(idx_hbm.at[wid], idx_v)
    copies = [
        pltpu.async_copy(
            table_hbm.at[idx_v.at[r]],
            rows_v.at[pl.ds(r * RL, RL)],
            sem,
        )
        for r in range(ROWS)
    ]
    for cp in copies:
        cp.wait()
    pltpu.sync_copy(rows_v, out_hbm.at[pl.ds(base, EPW)])


@functools.lru_cache(maxsize=None)
def _make_gather():
    return pl.kernel(
        _gather_body,
        out_type=jax.ShapeDtypeStruct((EPH, DPB), jnp.bfloat16),
        mesh=_mesh(),
        compiler_params=pltpu.CompilerParams(use_tc_tiling_on_sc=False),
        scratch_types=[
            pltpu.VMEM((ROWS, RL), jnp.int32),
            pltpu.VMEM((EPW, DPB), jnp.bfloat16),
            pltpu.SemaphoreType.DMA,
        ],
    )


def _gather_call(table, idx3):
    return _make_gather()(table, idx3)


def _scatter_body(msg_hbm, dst_hbm, zero_hbm, out_hbm,
                  idx_v, rows_v, stripe_v, acc_sh, sem):
    cid = lax.axis_index("c")
    sid = lax.axis_index("s")
    wid = sid * NC + cid
    base = wid * EPW
    pltpu.sync_copy(zero_hbm, stripe_v)
    pltpu.sync_copy(stripe_v, acc_sh.at[pl.ds(sid * NSTRIPE, NSTRIPE)])
    pltpu.sync_copy(dst_hbm.at[wid], idx_v)
    plsc.subcore_barrier()
    for c in range(NCH):
        pltpu.sync_copy(
            msg_hbm.at[pl.ds(base + c * GR * RL, GR * RL)], rows_v
        )
        for r in range(GR):
            pltpu.sync_copy(
                rows_v.at[pl.ds(r * RL, RL)],
                acc_sh.at[idx_v.at[c * GR + r]],
                add=True,
            )
    plsc.subcore_barrier()
    pltpu.sync_copy(acc_sh.at[pl.ds(sid * NSTRIPE, NSTRIPE)], stripe_v)
    pltpu.sync_copy(stripe_v, out_hbm.at[cid, pl.ds(sid * NSTRIPE, NSTRIPE)])


@functools.lru_cache(maxsize=None)
def _make_scatter():
    return pl.kernel(
        _scatter_body,
        out_type=jax.ShapeDtypeStruct((NC, NP, DP), jnp.float32),
        mesh=_mesh(),
        compiler_params=pltpu.CompilerParams(use_tc_tiling_on_sc=False),
        scratch_types=[
            pltpu.VMEM((ROWS, RL), jnp.int32),
            pltpu.VMEM((GR * RL, DP), jnp.float32),
            pltpu.VMEM((NSTRIPE, DP), jnp.float32),
            pltpu.VMEM_SHARED((NP, DP), jnp.float32),
            pltpu.SemaphoreType.DMA,
        ],
    )


def _scatter_call(msg, dst3, zero_stripe):
    return _make_scatter()(msg, dst3, zero_stripe)


def _out0_kernel(x_ref, w_ref, b_ref, o_ref, ob_ref):
    i = pl.program_id(0)
    h = jnp.maximum(jnp.dot(x_ref[...], w_ref[...],
                            preferred_element_type=jnp.float32)
                    + b_ref[...], 0.0)
    row = lax.broadcasted_iota(jnp.int32, (TN, D), 0) + i * TN
    h = jnp.where(row < N, h, 0.0)
    o_ref[...] = jnp.concatenate(
        [h, jnp.zeros((TN, DP - D), jnp.float32)], axis=1)
    ob_ref[...] = jnp.concatenate(
        [h.astype(jnp.bfloat16),
         jnp.zeros((TN, DPB - D), jnp.bfloat16)], axis=1)


def _run_out0(xp, w0t, b0):
    return pl.pallas_call(
        _out0_kernel,
        grid=(NP // TN,),
        in_specs=[
            pl.BlockSpec((TN, D), lambda i: (i, 0)),
            pl.BlockSpec((D, D), lambda i: (0, 0)),
            pl.BlockSpec((1, D), lambda i: (0, 0)),
        ],
        out_specs=[
            pl.BlockSpec((TN, DP), lambda i: (i, 0)),
            pl.BlockSpec((TN, DPB), lambda i: (i, 0)),
        ],
        out_shape=[
            jax.ShapeDtypeStruct((NP, DP), jnp.float32),
            jax.ShapeDtypeStruct((NP, DPB), jnp.bfloat16),
        ],
    )(xp, w0t, b0)


def _msg_kernel(h_ref, ea_ref, we1t_ref, be1_ref, r1_ref, r2_ref,
                wb_ref, b48_ref, o_ref):
    z = jnp.maximum(jnp.dot(ea_ref[...], we1t_ref[...],
                            preferred_element_type=jnp.float32)
                    + be1_ref[...], 0.0)
    hb = h_ref[...]
    zb = z.astype(jnp.bfloat16)
    hrep = jnp.dot(hb, r1_ref[...], preferred_element_type=jnp.float32)
    zrep = jnp.dot(zb, r2_ref[...], preferred_element_type=jnp.float32)
    p = (hrep * zrep).astype(jnp.bfloat16)
    msg = (jnp.dot(p, wb_ref[...], preferred_element_type=jnp.float32)
           + jnp.dot(hb, b48_ref[...], preferred_element_type=jnp.float32))
    o_ref[...] = jnp.concatenate(
        [msg, jnp.zeros((TE, DP - D), jnp.float32)], axis=1)


def _run_msg(hsrc, eah, we1t, be1, r1, r2, wbig, b48):
    return pl.pallas_call(
        _msg_kernel,
        grid=(EPH // TE,),
        in_specs=[
            pl.BlockSpec((TE, DPB), lambda i: (i, 0)),
            pl.BlockSpec((TE, DE), lambda i: (i, 0)),
            pl.BlockSpec((DE, EH), lambda i: (0, 0)),
            pl.BlockSpec((1, EH), lambda i: (0, 0)),
            pl.BlockSpec((DPB, D * EH), lambda i: (0, 0)),
            pl.BlockSpec((EH, D * EH), lambda i: (0, 0)),
            pl.BlockSpec((D * EH, D), lambda i: (0, 0)),
            pl.BlockSpec((DPB, D), lambda i: (0, 0)),
        ],
        out_specs=pl.BlockSpec((TE, DP), lambda i: (i, 0)),
        out_shape=jax.ShapeDtypeStruct((EPH, DP), jnp.float32),
    )(hsrc, eah, we1t, be1, r1, r2, wbig, b48)


def _update_kernel(add_init, agga_ref, aggb_ref, prev_ref, cb_ref, wmt_ref,
                   bm_ref, init_ref, o_ref, ob_ref):
    i = pl.program_id(0)
    agg = (agga_ref[0][:, :D] + agga_ref[1][:, :D]
           + aggb_ref[0][:, :D] + aggb_ref[1][:, :D])
    prev = prev_ref[:, :D]
    m = jnp.maximum(agg + prev + cb_ref[...], 0.0)
    cat = jnp.concatenate([m, prev], axis=1)
    res = jnp.dot(cat, wmt_ref[...],
                  preferred_element_type=jnp.float32) + bm_ref[...]
    if add_init:
        res = res + init_ref[...]
    row = lax.broadcasted_iota(jnp.int32, (TN, D), 0) + i * TN
    res = jnp.where(row < N, res, 0.0)
    o_ref[...] = jnp.concatenate(
        [res, jnp.zeros((TN, DP - D), jnp.float32)], axis=1)
    ob_ref[...] = jnp.concatenate(
        [res.astype(jnp.bfloat16),
         jnp.zeros((TN, DPB - D), jnp.bfloat16)], axis=1)


def _run_update(agga, aggb, prev, cb, wmt, bm, xp, add_init):
    return pl.pallas_call(
        functools.partial(_update_kernel, add_init),
        grid=(NP // TN,),
        in_specs=[
            pl.BlockSpec((NC, TN, DP), lambda i: (0, i, 0)),
            pl.BlockSpec((NC, TN, DP), lambda i: (0, i, 0)),
            pl.BlockSpec((TN, DP), lambda i: (i, 0)),
            pl.BlockSpec((1, D), lambda i: (0, 0)),
            pl.BlockSpec((2 * D, D), lambda i: (0, 0)),
            pl.BlockSpec((1, D), lambda i: (0, 0)),
            pl.BlockSpec((TN, D), lambda i: (i, 0)),
        ],
        out_specs=[
            pl.BlockSpec((TN, DP), lambda i: (i, 0)),
            pl.BlockSpec((TN, DPB), lambda i: (i, 0)),
        ],
        out_shape=[
            jax.ShapeDtypeStruct((NP, DP), jnp.float32),
            jax.ShapeDtypeStruct((NP, DPB), jnp.bfloat16),
        ],
    )(agga, aggb, prev, cb, wmt, bm, xp)


def kernel(x, edge_index, edge_attr, W0, b0, We1, be1, We2, be2,
           conv_bias, Wm, bm):
    f32 = jnp.float32
    src = edge_index[0]
    dst = edge_index[1]

    wbig = We2.reshape(D, D, EH).transpose(0, 2, 1).reshape(D * EH, D)
    col = jnp.arange(D * EH, dtype=jnp.int32)
    r1 = (jnp.arange(DPB, dtype=jnp.int32)[:, None]
          == (col // EH)[None, :]).astype(jnp.bfloat16)
    r2 = (jnp.arange(EH, dtype=jnp.int32)[:, None]
          == (col % EH)[None, :]).astype(jnp.bfloat16)
    wbig = wbig.astype(jnp.bfloat16)
    b48 = jnp.pad(be2.reshape(D, D),
                  ((0, DPB - D), (0, 0))).astype(jnp.bfloat16)

    xp = jnp.pad(x, ((0, NP - N), (0, 0)))
    eap = jnp.pad(edge_attr, ((0, EP - E), (0, 0)))
    src4 = jnp.concatenate(
        [src, jnp.full((EP - E,), N, jnp.int32)]).reshape(NH, NW, ROWS, RL)
    dst4 = jnp.concatenate(
        [dst, jnp.zeros((EP - E,), jnp.int32)]).reshape(NH, NW, ROWS, RL)
    zero_stripe = jnp.zeros((NSTRIPE, DP), f32)

    w0t = W0.T
    we1t = We1.T
    wmt = Wm.T
    b0r = b0.reshape(1, D)
    be1r = be1.reshape(1, EH)
    cbr = conv_bias.reshape(1, D)
    bmr = bm.reshape(1, D)

    outp, outb = _run_out0(xp, w0t, b0r)
    for s in range(STEPS):
        hsrc_a = _gather_call(outb, src4[0])
        hsrc_b = _gather_call(outb, src4[1])
        msg_a = _run_msg(hsrc_a, eap[:EPH], we1t, be1r, r1, r2, wbig, b48)
        agg_a = _scatter_call(msg_a, dst4[0], zero_stripe)
        msg_b = _run_msg(hsrc_b, eap[EPH:], we1t, be1r, r1, r2, wbig, b48)
        agg_b = _scatter_call(msg_b, dst4[1], zero_stripe)
        outp, outb = _run_update(agg_a, agg_b, outp, cbr, wmt, bmr, xp,
                                 add_init=(s == STEPS - 1))
    return outp[:N, :D]


# speedup vs baseline: 1.0159x; 1.0159x over previous
import functools

import jax
import jax.numpy as jnp
from jax import lax
from jax.experimental import pallas as pl
from jax.experimental.pallas import tpu as pltpu
from jax.experimental.pallas import tpu_sc as plsc

N = 10000
E = 160000
D = 36
DE = 6
EH = 36
STEPS = 3

DP = 48
NP = 10240
EP = 163840
NH = 2
EPH = EP // NH
NC = 2
NS = 16
NW = NC * NS
RL = 128
EPW = EPH // NW
ROWS = EPW // RL
GR = 10
NCH = ROWS // GR
NSTRIPE = NP // NS

TE = 1280
TN = 640


@functools.lru_cache(maxsize=None)
def _mesh():
    return plsc.VectorSubcoreMesh(core_axis_name="c", subcore_axis_name="s")


EPWG = EP // NW
ROWSG = EPWG // RL
GW = 20


def _gather_body(table_hbm, idx_hbm, out_hbm, idx_v, rows_v, sem):
    cid = lax.axis_index("c")
    sid = lax.axis_index("s")
    wid = sid * NC + cid
    base = wid * EPWG
    pltpu.sync_copy(idx_hbm.at[wid], idx_v)
    for w in range(ROWSG // GW):
        copies = [
            pltpu.async_copy(
                table_hbm.at[idx_v.at[w * GW + r]],
                rows_v.at[pl.ds(r * RL, RL)],
                sem,
            )
            for r in range(GW)
        ]
        for cp in copies:
            cp.wait()
        pltpu.sync_copy(
            rows_v, out_hbm.at[pl.ds(base + w * GW * RL, GW * RL)])


@functools.lru_cache(maxsize=None)
def _make_gather():
    return pl.kernel(
        _gather_body,
        out_type=jax.ShapeDtypeStruct((EP, DP), jnp.float32),
        mesh=_mesh(),
        compiler_params=pltpu.CompilerParams(use_tc_tiling_on_sc=False),
        scratch_types=[
            pltpu.VMEM((ROWSG, RL), jnp.int32),
            pltpu.VMEM((GW * RL, DP), jnp.float32),
            pltpu.SemaphoreType.DMA,
        ],
    )


def _gather_call(table, idx3):
    return _make_gather()(table, idx3)


def _scatter_body(msg_hbm, dst_hbm, zero_hbm, out_hbm,
                  idx_v, rows_v, stripe_v, acc_sh, sem):
    cid = lax.axis_index("c")
    sid = lax.axis_index("s")
    wid = sid * NC + cid
    base = wid * EPW
    pltpu.sync_copy(zero_hbm, stripe_v)
    pltpu.sync_copy(stripe_v, acc_sh.at[pl.ds(sid * NSTRIPE, NSTRIPE)])
    pltpu.sync_copy(dst_hbm.at[wid], idx_v)
    plsc.subcore_barrier()
    for c in range(NCH):
        pltpu.sync_copy(
            msg_hbm.at[pl.ds(base + c * GR * RL, GR * RL)], rows_v
        )
        for r in range(GR):
            pltpu.sync_copy(
                rows_v.at[pl.ds(r * RL, RL)],
                acc_sh.at[idx_v.at[c * GR + r]],
                add=True,
            )
    plsc.subcore_barrier()
    pltpu.sync_copy(acc_sh.at[pl.ds(sid * NSTRIPE, NSTRIPE)], stripe_v)
    pltpu.sync_copy(stripe_v, out_hbm.at[cid, pl.ds(sid * NSTRIPE, NSTRIPE)])


@functools.lru_cache(maxsize=None)
def _make_scatter():
    return pl.kernel(
        _scatter_body,
        out_type=jax.ShapeDtypeStruct((NC, NP, DP), jnp.float32),
        mesh=_mesh(),
        compiler_params=pltpu.CompilerParams(use_tc_tiling_on_sc=False),
        scratch_types=[
            pltpu.VMEM((ROWS, RL), jnp.int32),
            pltpu.VMEM((GR * RL, DP), jnp.float32),
            pltpu.VMEM((NSTRIPE, DP), jnp.float32),
            pltpu.VMEM_SHARED((NP, DP), jnp.float32),
            pltpu.SemaphoreType.DMA,
        ],
    )


def _scatter_call(msg, dst3, zero_stripe):
    return _make_scatter()(msg, dst3, zero_stripe)


def _out0_kernel(x_ref, w_ref, b_ref, o_ref):
    i = pl.program_id(0)
    h = jnp.maximum(jnp.dot(x_ref[...], w_ref[...],
                            preferred_element_type=jnp.float32)
                    + b_ref[...], 0.0)
    row = lax.broadcasted_iota(jnp.int32, (TN, D), 0) + i * TN
    h = jnp.where(row < N, h, 0.0)
    o_ref[...] = jnp.concatenate(
        [h, jnp.zeros((TN, DP - D), jnp.float32)], axis=1)


def _run_out0(xp, w0t, b0):
    return pl.pallas_call(
        _out0_kernel,
        grid=(NP // TN,),
        in_specs=[
            pl.BlockSpec((TN, D), lambda i: (i, 0)),
            pl.BlockSpec((D, D), lambda i: (0, 0)),
            pl.BlockSpec((1, D), lambda i: (0, 0)),
        ],
        out_specs=pl.BlockSpec((TN, DP), lambda i: (i, 0)),
        out_shape=jax.ShapeDtypeStruct((NP, DP), jnp.float32),
    )(xp, w0t, b0)


def _msg_kernel(h_ref, ea_ref, we1t_ref, be1_ref, r1_ref, r2_ref,
                wb_ref, b48_ref, o_ref):
    z = jnp.maximum(jnp.dot(ea_ref[...], we1t_ref[...],
                            preferred_element_type=jnp.float32)
                    + be1_ref[...], 0.0)
    hb = h_ref[...].astype(jnp.bfloat16)
    zb = z.astype(jnp.bfloat16)
    hrep = jnp.dot(hb, r1_ref[...], preferred_element_type=jnp.float32)
    zrep = jnp.dot(zb, r2_ref[...], preferred_element_type=jnp.float32)
    p = (hrep * zrep).astype(jnp.bfloat16)
    msg = (jnp.dot(p, wb_ref[...], preferred_element_type=jnp.float32)
           + jnp.dot(hb, b48_ref[...], preferred_element_type=jnp.float32))
    o_ref[...] = jnp.concatenate(
        [msg, jnp.zeros((TE, DP - D), jnp.float32)], axis=1)


def _run_msg(half, hsrc, eap, we1t, be1, r1, r2, wbig, b48):
    off = half * (EPH // TE)
    return pl.pallas_call(
        _msg_kernel,
        grid=(EPH // TE,),
        in_specs=[
            pl.BlockSpec((TE, DP), lambda i: (i + off, 0)),
            pl.BlockSpec((TE, DE), lambda i: (i + off, 0)),
            pl.BlockSpec((DE, EH), lambda i: (0, 0)),
            pl.BlockSpec((1, EH), lambda i: (0, 0)),
            pl.BlockSpec((DP, D * EH), lambda i: (0, 0)),
            pl.BlockSpec((EH, D * EH), lambda i: (0, 0)),
            pl.BlockSpec((D * EH, D), lambda i: (0, 0)),
            pl.BlockSpec((DP, D), lambda i: (0, 0)),
        ],
        out_specs=pl.BlockSpec((TE, DP), lambda i: (i, 0)),
        out_shape=jax.ShapeDtypeStruct((EPH, DP), jnp.float32),
    )(hsrc, eap, we1t, be1, r1, r2, wbig, b48)


def _update_kernel(add_init, agga_ref, aggb_ref, prev_ref, cb_ref, wmt_ref,
                   bm_ref, init_ref, o_ref):
    i = pl.program_id(0)
    agg = (agga_ref[0][:, :D] + agga_ref[1][:, :D]
           + aggb_ref[0][:, :D] + aggb_ref[1][:, :D])
    prev = prev_ref[:, :D]
    m = jnp.maximum(agg + prev + cb_ref[...], 0.0)
    cat = jnp.concatenate([m, prev], axis=1)
    res = jnp.dot(cat, wmt_ref[...],
                  preferred_element_type=jnp.float32) + bm_ref[...]
    if add_init:
        res = res + init_ref[...]
    row = lax.broadcasted_iota(jnp.int32, (TN, D), 0) + i * TN
    res = jnp.where(row < N, res, 0.0)
    o_ref[...] = jnp.concatenate(
        [res, jnp.zeros((TN, DP - D), jnp.float32)], axis=1)


def _run_update(agga, aggb, prev, cb, wmt, bm, xp, add_init):
    return pl.pallas_call(
        functools.partial(_update_kernel, add_init),
        grid=(NP // TN,),
        in_specs=[
            pl.BlockSpec((NC, TN, DP), lambda i: (0, i, 0)),
            pl.BlockSpec((NC, TN, DP), lambda i: (0, i, 0)),
            pl.BlockSpec((TN, DP), lambda i: (i, 0)),
            pl.BlockSpec((1, D), lambda i: (0, 0)),
            pl.BlockSpec((2 * D, D), lambda i: (0, 0)),
            pl.BlockSpec((1, D), lambda i: (0, 0)),
            pl.BlockSpec((TN, D), lambda i: (i, 0)),
        ],
        out_specs=pl.BlockSpec((TN, DP), lambda i: (i, 0)),
        out_shape=jax.ShapeDtypeStruct((NP, DP), jnp.float32),
    )(agga, aggb, prev, cb, wmt, bm, xp)


def kernel(x, edge_index, edge_attr, W0, b0, We1, be1, We2, be2,
           conv_bias, Wm, bm):
    f32 = jnp.float32
    src = edge_index[0]
    dst = edge_index[1]

    wbig = We2.reshape(D, D, EH).transpose(0, 2, 1).reshape(D * EH, D)
    col = jnp.arange(D * EH, dtype=jnp.int32)
    r1 = (jnp.arange(DP, dtype=jnp.int32)[:, None]
          == (col // EH)[None, :]).astype(jnp.bfloat16)
    r2 = (jnp.arange(EH, dtype=jnp.int32)[:, None]
          == (col % EH)[None, :]).astype(jnp.bfloat16)
    wbig = wbig.astype(jnp.bfloat16)
    b48 = jnp.pad(be2.reshape(D, D),
                  ((0, DP - D), (0, 0))).astype(jnp.bfloat16)

    xp = jnp.pad(x, ((0, NP - N), (0, 0)))
    eap = jnp.pad(edge_attr, ((0, EP - E), (0, 0)))
    src3 = jnp.concatenate(
        [src, jnp.full((EP - E,), N, jnp.int32)]).reshape(NW, ROWSG, RL)
    dst4 = jnp.concatenate(
        [dst, jnp.zeros((EP - E,), jnp.int32)]).reshape(NH, NW, ROWS, RL)
    zero_stripe = jnp.zeros((NSTRIPE, DP), f32)

    w0t = W0.T
    we1t = We1.T
    wmt = Wm.T
    b0r = b0.reshape(1, D)
    be1r = be1.reshape(1, EH)
    cbr = conv_bias.reshape(1, D)
    bmr = bm.reshape(1, D)

    outp = _run_out0(xp, w0t, b0r)
    for s in range(STEPS):
        hsrc = _gather_call(outp, src3)
        msg_a = _run_msg(0, hsrc, eap, we1t, be1r, r1, r2, wbig, b48)
        agg_a = _scatter_call(msg_a, dst4[0], zero_stripe)
        msg_b = _run_msg(1, hsrc, eap, we1t, be1r, r1, r2, wbig, b48)
        agg_b = _scatter_call(msg_b, dst4[1], zero_stripe)
        outp = _run_update(agg_a, agg_b, outp, cbr, wmt, bmr, xp,
                           add_init=(s == STEPS - 1))
    return outp[:N, :D]


# speedup vs baseline: 1.1982x; 1.1794x over previous
import functools

import jax
import jax.numpy as jnp
from jax import lax
from jax.experimental import pallas as pl
from jax.experimental.pallas import tpu as pltpu
from jax.experimental.pallas import tpu_sc as plsc

N = 10000
E = 160000
D = 36
DE = 6
EH = 36
STEPS = 3

DP = 128
NP = 10240
EP = 163840
NH = 2
EPH = EP // NH
NC = 2
NS = 16
NW = NC * NS
RL = 128
EPW = EPH // NW
ROWS = EPW // RL
GW = 5
NSTRIPE = NP // NS
RC = 128

TE = 1280
TN = 640


@functools.lru_cache(maxsize=None)
def _mesh():
    return plsc.VectorSubcoreMesh(core_axis_name="c", subcore_axis_name="s")


def _gather_body(table_hbm, idx_hbm, out_hbm, idx_v, rows_v, sem):
    cid = lax.axis_index("c")
    sid = lax.axis_index("s")
    wid = sid * NC + cid
    base = wid * EPW
    pltpu.sync_copy(idx_hbm.at[wid], idx_v)
    for w in range(ROWS // GW):
        copies = [
            pltpu.async_copy(
                table_hbm.at[idx_v.at[w * GW + r]],
                rows_v.at[pl.ds(r * RL, RL)],
                sem,
            )
            for r in range(GW)
        ]
        for cp in copies:
            cp.wait()
        pltpu.sync_copy(
            rows_v, out_hbm.at[pl.ds(base + w * GW * RL, GW * RL)])


@functools.lru_cache(maxsize=None)
def _make_gather():
    return pl.kernel(
        _gather_body,
        out_type=jax.ShapeDtypeStruct((EPH, DP), jnp.float32),
        mesh=_mesh(),
        scratch_types=[
            pltpu.VMEM((ROWS, RL), jnp.int32),
            pltpu.VMEM((GW * RL, DP), jnp.float32),
            pltpu.SemaphoreType.DMA,
        ],
    )


def _gather_call(table, idx3):
    return _make_gather()(table, idx3)


def _scatter_body(msg_hbm, dst_hbm, zero_hbm, out_hbm,
                  idx_v, rows_v, acc_sh, sem):
    cid = lax.axis_index("c")
    sid = lax.axis_index("s")
    wid = sid * NC + cid
    base = wid * EPW
    pltpu.sync_copy(zero_hbm, rows_v)
    for j in range(NSTRIPE // RC):
        pltpu.sync_copy(
            rows_v,
            acc_sh.at[pl.ds(sid * NSTRIPE + j * RC, RC)])
    pltpu.sync_copy(dst_hbm.at[wid], idx_v)
    plsc.subcore_barrier()
    for c in range(ROWS):
        pltpu.sync_copy(msg_hbm.at[pl.ds(base + c * RL, RL)], rows_v)
        pltpu.sync_copy(rows_v, acc_sh.at[idx_v.at[c]], add=True)
    plsc.subcore_barrier()
    for j in range(NSTRIPE // RC):
        pltpu.sync_copy(
            acc_sh.at[pl.ds(sid * NSTRIPE + j * RC, RC)], rows_v)
        pltpu.sync_copy(
            rows_v, out_hbm.at[cid, pl.ds(sid * NSTRIPE + j * RC, RC)])


@functools.lru_cache(maxsize=None)
def _make_scatter():
    return pl.kernel(
        _scatter_body,
        out_type=jax.ShapeDtypeStruct((NC, NP, DP), jnp.float32),
        mesh=_mesh(),
        scratch_types=[
            pltpu.VMEM((ROWS, RL), jnp.int32),
            pltpu.VMEM((RC, DP), jnp.float32),
            pltpu.VMEM_SHARED((NP, DP), jnp.float32),
            pltpu.SemaphoreType.DMA,
        ],
    )


def _scatter_call(msg, dst3, zero_chunk):
    return _make_scatter()(msg, dst3, zero_chunk)


def _out0_kernel(x_ref, w_ref, b_ref, o_ref):
    i = pl.program_id(0)
    h = jnp.maximum(jnp.dot(x_ref[...], w_ref[...],
                            preferred_element_type=jnp.float32)
                    + b_ref[...], 0.0)
    row = lax.broadcasted_iota(jnp.int32, (TN, D), 0) + i * TN
    h = jnp.where(row < N, h, 0.0)
    o_ref[...] = jnp.concatenate(
        [h, jnp.zeros((TN, DP - D), jnp.float32)], axis=1)


def _run_out0(xp, w0t, b0):
    return pl.pallas_call(
        _out0_kernel,
        grid=(NP // TN,),
        in_specs=[
            pl.BlockSpec((TN, D), lambda i: (i, 0)),
            pl.BlockSpec((D, D), lambda i: (0, 0)),
            pl.BlockSpec((1, D), lambda i: (0, 0)),
        ],
        out_specs=pl.BlockSpec((TN, DP), lambda i: (i, 0)),
        out_shape=jax.ShapeDtypeStruct((NP, DP), jnp.float32),
    )(xp, w0t, b0)


def _msg_kernel(h_ref, ea_ref, we1t_ref, be1_ref, r1_ref, r2_ref,
                wb_ref, b128_ref, o_ref):
    z = jnp.maximum(jnp.dot(ea_ref[...], we1t_ref[...],
                            preferred_element_type=jnp.float32)
                    + be1_ref[...], 0.0)
    hb = h_ref[...].astype(jnp.bfloat16)
    zb = z.astype(jnp.bfloat16)
    hrep = jnp.dot(hb, r1_ref[...], preferred_element_type=jnp.float32)
    zrep = jnp.dot(zb, r2_ref[...], preferred_element_type=jnp.float32)
    p = (hrep * zrep).astype(jnp.bfloat16)
    msg = (jnp.dot(p, wb_ref[...], preferred_element_type=jnp.float32)
           + jnp.dot(hb, b128_ref[...], preferred_element_type=jnp.float32))
    o_ref[...] = jnp.concatenate(
        [msg, jnp.zeros((TE, DP - D), jnp.float32)], axis=1)


def _run_msg(hsrc, eah, we1t, be1, r1, r2, wbig, b128):
    return pl.pallas_call(
        _msg_kernel,
        grid=(EPH // TE,),
        in_specs=[
            pl.BlockSpec((TE, DP), lambda i: (i, 0)),
            pl.BlockSpec((TE, DE), lambda i: (i, 0)),
            pl.BlockSpec((DE, EH), lambda i: (0, 0)),
            pl.BlockSpec((1, EH), lambda i: (0, 0)),
            pl.BlockSpec((DP, D * EH), lambda i: (0, 0)),
            pl.BlockSpec((EH, D * EH), lambda i: (0, 0)),
            pl.BlockSpec((D * EH, D), lambda i: (0, 0)),
            pl.BlockSpec((DP, D), lambda i: (0, 0)),
        ],
        out_specs=pl.BlockSpec((TE, DP), lambda i: (i, 0)),
        out_shape=jax.ShapeDtypeStruct((EPH, DP), jnp.float32),
    )(hsrc, eah, we1t, be1, r1, r2, wbig, b128)


def _update_kernel(add_init, agga_ref, aggb_ref, prev_ref, cb_ref, wmt_ref,
                   bm_ref, init_ref, o_ref):
    i = pl.program_id(0)
    agg = (agga_ref[0][:, :D] + agga_ref[1][:, :D]
           + aggb_ref[0][:, :D] + aggb_ref[1][:, :D])
    prev = prev_ref[:, :D]
    m = jnp.maximum(agg + prev + cb_ref[...], 0.0)
    cat = jnp.concatenate([m, prev], axis=1)
    res = jnp.dot(cat, wmt_ref[...],
                  preferred_element_type=jnp.float32) + bm_ref[...]
    if add_init:
        res = res + init_ref[...]
    row = lax.broadcasted_iota(jnp.int32, (TN, D), 0) + i * TN
    res = jnp.where(row < N, res, 0.0)
    o_ref[...] = jnp.concatenate(
        [res, jnp.zeros((TN, DP - D), jnp.float32)], axis=1)


def _run_update(agga, aggb, prev, cb, wmt, bm, xp, add_init):
    return pl.pallas_call(
        functools.partial(_update_kernel, add_init),
        grid=(NP // TN,),
        in_specs=[
            pl.BlockSpec((NC, TN, DP), lambda i: (0, i, 0)),
            pl.BlockSpec((NC, TN, DP), lambda i: (0, i, 0)),
            pl.BlockSpec((TN, DP), lambda i: (i, 0)),
            pl.BlockSpec((1, D), lambda i: (0, 0)),
            pl.BlockSpec((2 * D, D), lambda i: (0, 0)),
            pl.BlockSpec((1, D), lambda i: (0, 0)),
            pl.BlockSpec((TN, D), lambda i: (i, 0)),
        ],
        out_specs=pl.BlockSpec((TN, DP), lambda i: (i, 0)),
        out_shape=jax.ShapeDtypeStruct((NP, DP), jnp.float32),
    )(agga, aggb, prev, cb, wmt, bm, xp)


def kernel(x, edge_index, edge_attr, W0, b0, We1, be1, We2, be2,
           conv_bias, Wm, bm):
    f32 = jnp.float32
    src = edge_index[0]
    dst = edge_index[1]

    wbig = We2.reshape(D, D, EH).transpose(0, 2, 1).reshape(D * EH, D)
    col = jnp.arange(D * EH, dtype=jnp.int32)
    r1 = (jnp.arange(DP, dtype=jnp.int32)[:, None]
          == (col // EH)[None, :]).astype(jnp.bfloat16)
    r2 = (jnp.arange(EH, dtype=jnp.int32)[:, None]
          == (col % EH)[None, :]).astype(jnp.bfloat16)
    wbig = wbig.astype(jnp.bfloat16)
    b128 = jnp.pad(be2.reshape(D, D),
                   ((0, DP - D), (0, 0))).astype(jnp.bfloat16)

    xp = jnp.pad(x, ((0, NP - N), (0, 0)))
    eap = jnp.pad(edge_attr, ((0, EP - E), (0, 0)))
    src4 = jnp.concatenate(
        [src, jnp.full((EP - E,), N, jnp.int32)]).reshape(NH, NW, ROWS, RL)
    dst4 = jnp.concatenate(
        [dst, jnp.zeros((EP - E,), jnp.int32)]).reshape(NH, NW, ROWS, RL)
    zero_chunk = jnp.zeros((RC, DP), f32)

    w0t = W0.T
    we1t = We1.T
    wmt = Wm.T
    b0r = b0.reshape(1, D)
    be1r = be1.reshape(1, EH)
    cbr = conv_bias.reshape(1, D)
    bmr = bm.reshape(1, D)

    outp = _run_out0(xp, w0t, b0r)
    for s in range(STEPS):
        hsrc_a = _gather_call(outp, src4[0])
        hsrc_b = _gather_call(outp, src4[1])
        msg_a = _run_msg(hsrc_a, eap[:EPH], we1t, be1r, r1, r2, wbig, b128)
        agg_a = _scatter_call(msg_a, dst4[0], zero_chunk)
        msg_b = _run_msg(hsrc_b, eap[EPH:], we1t, be1r, r1, r2, wbig, b128)
        agg_b = _scatter_call(msg_b, dst4[1], zero_chunk)
        outp = _run_update(agg_a, agg_b, outp, cbr, wmt, bmr, xp,
                           add_init=(s == STEPS - 1))
    return outp[:N, :D]
